# baseline (speedup 1.0000x reference)
import jax
import jax.numpy as jnp
from jax.experimental import pallas as pl
from jax.experimental.pallas import tpu as pltpu

_N = 4096
_F = 128
_C = 64
_BV = 512
_NV = _N // _BV

_DN = (((1,), (0,)), ((), ()))


def _mm(a, b):
    return jax.lax.dot_general(a, b, _DN, preferred_element_type=jnp.float32)


def _body(hcatT_ref, adj_ref, w0, b0, w1, b1, w12, b12, ws2, b2,
          out_ref, adj_scr, ideg_scr, x1_scr, x2_scr, y2_scr):
    s = pl.program_id(0)
    v = pl.program_id(1)
    cols = pl.ds(v * _BV, _BV)

    @pl.when(s == 0)
    def _stage0():
        ab = adj_ref[...].astype(jnp.bfloat16)
        adj_scr[:, cols] = ab
        zT = _mm(hcatT_ref[...], ab)
        ideg = 1.0 / jnp.maximum(zT[_F:_F + 1, :], 1.0)
        ideg_scr[:, cols] = ideg
        zs = (zT[:_F, :] * ideg).astype(jnp.bfloat16)
        x1T = (_mm(w0[:, :_F], hcatT_ref[:_F, cols])
               + _mm(w0[:, _F:], zs) + b0[...])
        x1_scr[:, cols] = x1T.astype(jnp.bfloat16)

    @pl.when(s == 1)
    def _stage1():
        ab = adj_scr[:, cols]
        zT = _mm(x1_scr[...], ab)
        zs = (zT * ideg_scr[:, cols]).astype(jnp.bfloat16)
        x1b = x1_scr[:, cols]
        x2T = _mm(w1[:, :_F], x1b) + _mm(w1[:, _F:], zs) + b1[...]
        x2_scr[:, cols] = x2T.astype(jnp.bfloat16)
        y2T = (_mm(w12[:, :_F], x1b) + _mm(w12[:, _F:], zs)
               + b12[...])
        y2_scr[:, cols] = y2T.astype(jnp.bfloat16)

    @pl.when(s == 2)
    def _stage2():
        ab = adj_scr[:, cols]
        zT = _mm(y2_scr[...], ab)
        zs = zT * ideg_scr[:, cols]
        out_ref[...] = _mm(ws2[...], x2_scr[:, cols]) + zs + b2[...]


def kernel(h, adj, W_self0, W_neigh0, b0, W_self1, W_neigh1, b1,
           W_self2, W_neigh2, b2):
    bf = jnp.bfloat16
    hcatT = jnp.concatenate(
        [h.T.astype(bf), jnp.ones((8, _N), bf)], axis=0)
    w0 = jnp.concatenate([W_self0, W_neigh0], axis=1).astype(bf)
    w1cat = jnp.concatenate([W_self1, W_neigh1], axis=1)
    w1 = w1cat.astype(bf)
    w12 = (W_neigh2 @ w1cat).astype(bf)
    b12 = (W_neigh2 @ b1).reshape(-1, 1)
    full = lambda shape: pl.BlockSpec(shape, lambda s, v: (0, 0))
    outT = pl.pallas_call(
        _body,
        grid=(3, _NV),
        in_specs=[
            full((_F + 8, _N)),
            pl.BlockSpec((_N, _BV),
                         lambda s, v: (0, jnp.where(s == 0, v, _NV - 1))),
            full((_F, 2 * _F)), full((_F, 1)),
            full((_F, 2 * _F)), full((_F, 1)),
            full((_C, 2 * _F)), full((_C, 1)),
            full((_C, _F)), full((_C, 1)),
        ],
        out_specs=pl.BlockSpec((_C, _BV),
                               lambda s, v: (0, jnp.where(s == 2, v, 0))),
        out_shape=jax.ShapeDtypeStruct((_C, _N), jnp.float32),
        scratch_shapes=[
            pltpu.VMEM((_N, _N), jnp.bfloat16),
            pltpu.VMEM((1, _N), jnp.float32),
            pltpu.VMEM((_F, _N), jnp.bfloat16),
            pltpu.VMEM((_F, _N), jnp.bfloat16),
            pltpu.VMEM((_C, _N), jnp.bfloat16),
        ],
        compiler_params=pltpu.CompilerParams(
            dimension_semantics=("arbitrary", "arbitrary"),
            vmem_limit_bytes=128 * 1024 * 1024,
        ),
    )(hcatT, adj, w0, b0.reshape(-1, 1), w1, b1.reshape(-1, 1),
      w12, b12, W_self2.astype(bf), b2.reshape(-1, 1))
    return outT.T


# speedup vs baseline: 1.2079x; 1.2079x over previous
import jax
import jax.numpy as jnp
from jax.experimental import pallas as pl
from jax.experimental.pallas import tpu as pltpu

_N = 4096
_F = 128
_C = 64
_B0 = 512
_NB0 = _N // _B0
_B12 = 1024
_NB12 = _N // _B12
_T1 = _NB0
_T2 = _NB0 + _NB12

_DN = (((1,), (0,)), ((), ()))


def _mm(a, b):
    return jax.lax.dot_general(a, b, _DN, preferred_element_type=jnp.float32)


def _body(hT_ref, hcatT_ref, adj_ref, ws0, wn0, b0, ws1, wn1, b1, ws2, wn2, b2,
          out_ref, adj_scr, ideg_scr, x1T_scr, x1b_scr, x2T_scr, y2b_scr):
    t = pl.program_id(0)

    @pl.when(t < _T1)
    def _stage0():
        cols = pl.ds(t * _B0, _B0)
        ab = adj_ref[...].astype(jnp.bfloat16)
        adj_scr[:, cols] = ab
        zT = _mm(hcatT_ref[...], ab)
        ideg = 1.0 / jnp.maximum(zT[_F:_F + 1, :], 1.0)
        ideg_scr[:, cols] = ideg
        zs = zT[:_F, :] * ideg
        x1T = _mm(ws0[...], hT_ref[:, cols]) + _mm(wn0[...], zs) + b0[...]
        x1T_scr[:, cols] = x1T
        x1b_scr[:, cols] = x1T.astype(jnp.bfloat16)

    @pl.when(jnp.logical_and(t >= _T1, t < _T2))
    def _stage1():
        cols = pl.ds((t - _T1) * _B12, _B12)
        zT = _mm(x1b_scr[...], adj_scr[:, cols])
        zs = zT * ideg_scr[:, cols]
        x2T = _mm(ws1[...], x1T_scr[:, cols]) + _mm(wn1[...], zs) + b1[...]
        x2T_scr[:, cols] = x2T
        y2T = _mm(wn2[...], x2T)
        y2b_scr[:, cols] = y2T.astype(jnp.bfloat16)

    @pl.when(t >= _T2)
    def _stage2():
        cols = pl.ds((t - _T2) * _B12, _B12)
        zT = _mm(y2b_scr[...], adj_scr[:, cols])
        zs = zT * ideg_scr[:, cols]
        out_ref[...] = _mm(ws2[...], x2T_scr[:, cols]) + zs + b2[...]


def kernel(h, adj, W_self0, W_neigh0, b0, W_self1, W_neigh1, b1,
           W_self2, W_neigh2, b2):
    hT = h.T
    hcatT = jnp.concatenate(
        [hT.astype(jnp.bfloat16), jnp.ones((8, _N), jnp.bfloat16)], axis=0)
    full = lambda shape: pl.BlockSpec(shape, lambda t: (0, 0))
    outT = pl.pallas_call(
        _body,
        grid=(_T2 + _NB12,),
        in_specs=[
            full((_F, _N)),
            full((_F + 8, _N)),
            pl.BlockSpec((_N, _B0),
                         lambda t: (0, jnp.where(t < _T1, t, _T1 - 1))),
            full((_F, _F)), full((_F, _F)), full((_F, 1)),
            full((_F, _F)), full((_F, _F)), full((_F, 1)),
            full((_C, _F)), full((_C, _F)), full((_C, 1)),
        ],
        out_specs=pl.BlockSpec(
            (_C, _B12), lambda t: (0, jnp.where(t >= _T2, t - _T2, 0))),
        out_shape=jax.ShapeDtypeStruct((_C, _N), jnp.float32),
        scratch_shapes=[
            pltpu.VMEM((_N, _N), jnp.bfloat16),
            pltpu.VMEM((1, _N), jnp.float32),
            pltpu.VMEM((_F, _N), jnp.float32),
            pltpu.VMEM((_F, _N), jnp.bfloat16),
            pltpu.VMEM((_F, _N), jnp.float32),
            pltpu.VMEM((_C, _N), jnp.bfloat16),
        ],
        compiler_params=pltpu.CompilerParams(
            dimension_semantics=("arbitrary",),
            vmem_limit_bytes=128 * 1024 * 1024,
        ),
    )(hT, hcatT, adj, W_self0, W_neigh0, b0.reshape(-1, 1),
      W_self1, W_neigh1, b1.reshape(-1, 1),
      W_self2, W_neigh2, b2.reshape(-1, 1))
    return outT.T


# speedup vs baseline: 1.2476x; 1.0328x over previous
import jax
import jax.numpy as jnp
from jax.experimental import pallas as pl
from jax.experimental.pallas import tpu as pltpu

_N = 4096
_F = 128
_C = 64
_B0 = 512
_NB0 = _N // _B0
_B12 = 2048
_NB12 = _N // _B12
_T1 = _NB0
_T2 = _NB0 + _NB12

_DN = (((1,), (0,)), ((), ()))


def _mm(a, b):
    return jax.lax.dot_general(a, b, _DN, preferred_element_type=jnp.float32)


def _body(hT_ref, hcatT_ref, adj_ref, ws0, wn0, b0, ws1, wn1, b1, ws2, wn2, b2,
          out_ref, adj_scr, ideg_scr, x1T_scr, x1b_scr, x2T_scr, y2b_scr):
    t = pl.program_id(0)

    @pl.when(t < _T1)
    def _stage0():
        cols = pl.ds(t * _B0, _B0)
        ab = adj_ref[...].astype(jnp.bfloat16)
        adj_scr[:, cols] = ab
        zT = _mm(hcatT_ref[...], ab)
        ideg = 1.0 / jnp.maximum(zT[_F:_F + 1, :], 1.0)
        ideg_scr[:, cols] = ideg
        zs = zT[:_F, :] * ideg
        x1T = _mm(ws0[...], hT_ref[:, cols]) + _mm(wn0[...], zs) + b0[...]
        x1T_scr[:, cols] = x1T
        x1b_scr[:, cols] = x1T.astype(jnp.bfloat16)

    @pl.when(jnp.logical_and(t >= _T1, t < _T2))
    def _stage1():
        cols = pl.ds((t - _T1) * _B12, _B12)
        zT = _mm(x1b_scr[...], adj_scr[:, cols])
        zs = zT * ideg_scr[:, cols]
        x2T = _mm(ws1[...], x1T_scr[:, cols]) + _mm(wn1[...], zs) + b1[...]
        x2T_scr[:, cols] = x2T
        y2T = _mm(wn2[...], x2T)
        y2b_scr[:, cols] = y2T.astype(jnp.bfloat16)

    @pl.when(t >= _T2)
    def _stage2():
        cols = pl.ds((t - _T2) * _B12, _B12)
        zT = _mm(y2b_scr[...], adj_scr[:, cols])
        zs = zT * ideg_scr[:, cols]
        out_ref[...] = _mm(ws2[...], x2T_scr[:, cols]) + zs + b2[...]


def kernel(h, adj, W_self0, W_neigh0, b0, W_self1, W_neigh1, b1,
           W_self2, W_neigh2, b2):
    hT = h.T
    hcatT = jnp.concatenate(
        [hT.astype(jnp.bfloat16), jnp.ones((8, _N), jnp.bfloat16)], axis=0)
    full = lambda shape: pl.BlockSpec(shape, lambda t: (0, 0))
    outT = pl.pallas_call(
        _body,
        grid=(_T2 + _NB12,),
        in_specs=[
            full((_F, _N)),
            full((_F + 8, _N)),
            pl.BlockSpec((_N, _B0),
                         lambda t: (0, jnp.where(t < _T1, t, _T1 - 1))),
            full((_F, _F)), full((_F, _F)), full((_F, 1)),
            full((_F, _F)), full((_F, _F)), full((_F, 1)),
            full((_C, _F)), full((_C, _F)), full((_C, 1)),
        ],
        out_specs=pl.BlockSpec(
            (_C, _B12), lambda t: (0, jnp.where(t >= _T2, t - _T2, 0))),
        out_shape=jax.ShapeDtypeStruct((_C, _N), jnp.float32),
        scratch_shapes=[
            pltpu.VMEM((_N, _N), jnp.bfloat16),
            pltpu.VMEM((1, _N), jnp.float32),
            pltpu.VMEM((_F, _N), jnp.float32),
            pltpu.VMEM((_F, _N), jnp.bfloat16),
            pltpu.VMEM((_F, _N), jnp.float32),
            pltpu.VMEM((_C, _N), jnp.bfloat16),
        ],
        compiler_params=pltpu.CompilerParams(
            dimension_semantics=("arbitrary",),
            vmem_limit_bytes=128 * 1024 * 1024,
        ),
    )(hT, hcatT, adj, W_self0, W_neigh0, b0.reshape(-1, 1),
      W_self1, W_neigh1, b1.reshape(-1, 1),
      W_self2, W_neigh2, b2.reshape(-1, 1))
    return outT.T


# speedup vs baseline: 1.2701x; 1.0180x over previous
import jax
import jax.numpy as jnp
from jax.experimental import pallas as pl
from jax.experimental.pallas import tpu as pltpu

_N = 4096
_F = 128
_C = 64
_B0 = 512
_NB0 = _N // _B0
_B12 = 4096
_NB12 = _N // _B12
_T1 = _NB0
_T2 = _NB0 + _NB12

_DN = (((1,), (0,)), ((), ()))


def _mm(a, b):
    return jax.lax.dot_general(a, b, _DN, preferred_element_type=jnp.float32)


def _body(hT_ref, hcatT_ref, adj_ref, ws0, wn0, b0, ws1, wn1, b1, ws2, wn2, b2,
          out_ref, adj_scr, ideg_scr, x1T_scr, x1b_scr, x2T_scr, y2b_scr):
    t = pl.program_id(0)

    @pl.when(t < _T1)
    def _stage0():
        cols = pl.ds(t * _B0, _B0)
        ab = adj_ref[...].astype(jnp.bfloat16)
        adj_scr[:, cols] = ab
        zT = _mm(hcatT_ref[...], ab)
        ideg = 1.0 / jnp.maximum(zT[_F:_F + 1, :], 1.0)
        ideg_scr[:, cols] = ideg
        zs = zT[:_F, :] * ideg
        x1T = _mm(ws0[...], hT_ref[:, cols]) + _mm(wn0[...], zs) + b0[...]
        x1T_scr[:, cols] = x1T
        x1b_scr[:, cols] = x1T.astype(jnp.bfloat16)

    @pl.when(jnp.logical_and(t >= _T1, t < _T2))
    def _stage1():
        cols = pl.ds((t - _T1) * _B12, _B12)
        zT = _mm(x1b_scr[...], adj_scr[:, cols])
        zs = zT * ideg_scr[:, cols]
        x2T = _mm(ws1[...], x1T_scr[:, cols]) + _mm(wn1[...], zs) + b1[...]
        x2T_scr[:, cols] = x2T
        y2T = _mm(wn2[...], x2T)
        y2b_scr[:, cols] = y2T.astype(jnp.bfloat16)

    @pl.when(t >= _T2)
    def _stage2():
        cols = pl.ds((t - _T2) * _B12, _B12)
        zT = _mm(y2b_scr[...], adj_scr[:, cols])
        zs = zT * ideg_scr[:, cols]
        out_ref[...] = _mm(ws2[...], x2T_scr[:, cols]) + zs + b2[...]


def kernel(h, adj, W_self0, W_neigh0, b0, W_self1, W_neigh1, b1,
           W_self2, W_neigh2, b2):
    hT = h.T
    hcatT = jnp.concatenate(
        [hT.astype(jnp.bfloat16), jnp.ones((8, _N), jnp.bfloat16)], axis=0)
    full = lambda shape: pl.BlockSpec(shape, lambda t: (0, 0))
    outT = pl.pallas_call(
        _body,
        grid=(_T2 + _NB12,),
        in_specs=[
            full((_F, _N)),
            full((_F + 8, _N)),
            pl.BlockSpec((_N, _B0),
                         lambda t: (0, jnp.where(t < _T1, t, _T1 - 1))),
            full((_F, _F)), full((_F, _F)), full((_F, 1)),
            full((_F, _F)), full((_F, _F)), full((_F, 1)),
            full((_C, _F)), full((_C, _F)), full((_C, 1)),
        ],
        out_specs=pl.BlockSpec(
            (_C, _B12), lambda t: (0, jnp.where(t >= _T2, t - _T2, 0))),
        out_shape=jax.ShapeDtypeStruct((_C, _N), jnp.float32),
        scratch_shapes=[
            pltpu.VMEM((_N, _N), jnp.bfloat16),
            pltpu.VMEM((1, _N), jnp.float32),
            pltpu.VMEM((_F, _N), jnp.float32),
            pltpu.VMEM((_F, _N), jnp.bfloat16),
            pltpu.VMEM((_F, _N), jnp.float32),
            pltpu.VMEM((_C, _N), jnp.bfloat16),
        ],
        compiler_params=pltpu.CompilerParams(
            dimension_semantics=("arbitrary",),
            vmem_limit_bytes=128 * 1024 * 1024,
        ),
    )(hT, hcatT, adj, W_self0, W_neigh0, b0.reshape(-1, 1),
      W_self1, W_neigh1, b1.reshape(-1, 1),
      W_self2, W_neigh2, b2.reshape(-1, 1))
    return outT.T


# speedup vs baseline: 1.2949x; 1.0196x over previous
import jax
import jax.numpy as jnp
from jax.experimental import pallas as pl
from jax.experimental.pallas import tpu as pltpu

_N = 4096
_F = 128
_C = 64
_B0 = 512
_NB0 = _N // _B0
_T1 = _NB0
_T2 = _NB0 + 1

_DN = (((1,), (0,)), ((), ()))


def _mm(a, b):
    return jax.lax.dot_general(a, b, _DN, preferred_element_type=jnp.float32)


def _body(h_ref, adj_ref, ws0, wn0, b0, ws1, wn1, b1, ws2, wn2, b2,
          out_ref, adj_scr, hcat_scr, ideg_scr, x1T_scr, x1b_scr, x2T_scr,
          y2b_scr):
    t = pl.program_id(0)

    @pl.when(t == 0)
    def _prologue():
        hb = h_ref[...].astype(jnp.bfloat16)
        hcat_scr[:_F, :] = hb.T
        hcat_scr[_F:, :] = jnp.ones((8, _N), jnp.bfloat16)

    @pl.when(t < _T1)
    def _stage0():
        cols = pl.ds(t * _B0, _B0)
        ab = adj_ref[...].astype(jnp.bfloat16)
        adj_scr[:, cols] = ab
        zT = _mm(hcat_scr[...], ab)
        ideg = 1.0 / jnp.maximum(zT[_F:_F + 1, :], 1.0)
        ideg_scr[:, cols] = ideg
        zs = zT[:_F, :] * ideg
        x1T = _mm(ws0[...], hcat_scr[:_F, cols]) + _mm(wn0[...], zs) + b0[...]
        x1T_scr[:, cols] = x1T
        x1b_scr[:, cols] = x1T.astype(jnp.bfloat16)

    @pl.when(t == _T1)
    def _stage1():
        zT = _mm(x1b_scr[...], adj_scr[...])
        zs = zT * ideg_scr[...]
        x2T = _mm(ws1[...], x1T_scr[...]) + _mm(wn1[...], zs) + b1[...]
        x2T_scr[...] = x2T
        y2T = _mm(wn2[...], x2T)
        y2b_scr[...] = y2T.astype(jnp.bfloat16)

    @pl.when(t >= _T2)
    def _stage2():
        zT = _mm(y2b_scr[...], adj_scr[...])
        zs = zT * ideg_scr[...]
        outT = _mm(ws2[...], x2T_scr[...]) + zs + b2[...]
        out_ref[...] = outT.T


def kernel(h, adj, W_self0, W_neigh0, b0, W_self1, W_neigh1, b1,
           W_self2, W_neigh2, b2):
    full = lambda shape: pl.BlockSpec(shape, lambda t: (0, 0))
    out = pl.pallas_call(
        _body,
        grid=(_T2 + 1,),
        in_specs=[
            full((_N, _F)),
            pl.BlockSpec((_N, _B0),
                         lambda t: (0, jnp.where(t < _T1, t, _T1 - 1))),
            full((_F, _F)), full((_F, _F)), full((_F, 1)),
            full((_F, _F)), full((_F, _F)), full((_F, 1)),
            full((_C, _F)), full((_C, _F)), full((_C, 1)),
        ],
        out_specs=full((_N, _C)),
        out_shape=jax.ShapeDtypeStruct((_N, _C), jnp.float32),
        scratch_shapes=[
            pltpu.VMEM((_N, _N), jnp.bfloat16),
            pltpu.VMEM((_F + 8, _N), jnp.bfloat16),
            pltpu.VMEM((1, _N), jnp.float32),
            pltpu.VMEM((_F, _N), jnp.float32),
            pltpu.VMEM((_F, _N), jnp.bfloat16),
            pltpu.VMEM((_F, _N), jnp.float32),
            pltpu.VMEM((_C, _N), jnp.bfloat16),
        ],
        compiler_params=pltpu.CompilerParams(
            dimension_semantics=("arbitrary",),
            vmem_limit_bytes=128 * 1024 * 1024,
        ),
    )(h, adj, W_self0, W_neigh0, b0.reshape(-1, 1),
      W_self1, W_neigh1, b1.reshape(-1, 1),
      W_self2, W_neigh2, b2.reshape(-1, 1))
    return out
